# Initial kernel scaffold; baseline (speedup 1.0000x reference)
import jax
import jax.numpy as jnp
from jax.experimental import pallas as pl


def kernel(x, head, choice, w1, w2, merge):
    raise NotImplementedError("write your pallas kernel here")


# speedup vs baseline: 5.3151x; 5.3151x over previous
import functools

import jax
import jax.numpy as jnp
from jax import lax
from jax.experimental import pallas as pl
from jax.experimental.pallas import tpu as pltpu
from jax.experimental.pallas import tpu_sc as plsc

D_MODEL = 1024
N_HEAD = 8
D_FFN = 512
N_EXP = 16
CAP = 2
D_H = D_MODEL // N_HEAD


def _stage1(x, head, choice):
    B, BLK, _ = x.shape
    S = BLK * N_HEAD
    TS = 256

    def body(x_ref, head_ref, choice_ref, h_ref, pT_ref):
        xb = x_ref[0]
        hb = jnp.dot(xb, head_ref[...], preferred_element_type=jnp.float32)
        h_ref[0] = hb
        ht = hb.reshape(TS * N_HEAD, D_H)
        lt = lax.dot_general(choice_ref[...], ht, (((1,), (1,)), ((), ())),
                             preferred_element_type=jnp.float32)
        m = jnp.max(lt, axis=0, keepdims=True)
        ex = jnp.exp(lt - m)
        pT_ref[0] = ex / jnp.sum(ex, axis=0, keepdims=True)

    return pl.pallas_call(
        body,
        grid=(B, BLK // TS),
        in_specs=[
            pl.BlockSpec((1, TS, D_MODEL), lambda b, i: (b, i, 0)),
            pl.BlockSpec((D_MODEL, D_MODEL), lambda b, i: (0, 0)),
            pl.BlockSpec((N_EXP, D_H), lambda b, i: (0, 0)),
        ],
        out_specs=[
            pl.BlockSpec((1, TS, D_MODEL), lambda b, i: (b, i, 0)),
            pl.BlockSpec((1, N_EXP, TS * N_HEAD), lambda b, i: (b, 0, i)),
        ],
        out_shape=[
            jax.ShapeDtypeStruct((B, BLK, D_MODEL), jnp.float32),
            jax.ShapeDtypeStruct((B, N_EXP, S), jnp.float32),
        ],
    )(x, head, choice)


def _tc_mlp(xin, w1, w2, g):
    B, E, K, _ = xin.shape

    def body(x_ref, w1_ref, w2_ref, g_ref, o_ref):
        xb = x_ref[0, 0]
        a = jnp.dot(xb, w1_ref[0], preferred_element_type=jnp.float32)
        h1 = a * (1.0 / (1.0 + jnp.exp(-a)))
        y = jnp.dot(h1, w2_ref[0], preferred_element_type=jnp.float32)
        o_ref[0, 0] = y * g_ref[0, 0]

    return pl.pallas_call(
        body,
        grid=(B, E),
        in_specs=[
            pl.BlockSpec((1, 1, K, D_H), lambda b, e: (b, e, 0, 0)),
            pl.BlockSpec((1, D_H, D_FFN), lambda b, e: (e, 0, 0)),
            pl.BlockSpec((1, D_FFN, D_H), lambda b, e: (e, 0, 0)),
            pl.BlockSpec((1, 1, K, 1), lambda b, e: (b, e, 0, 0)),
        ],
        out_specs=pl.BlockSpec((1, 1, K, D_H), lambda b, e: (b, e, 0, 0)),
        out_shape=jax.ShapeDtypeStruct((B, E, K, D_H), jnp.float32),
    )(xin, w1, w2, g)


def _tc_merge(y, merge):
    B, BLK, _ = y.shape
    TS = 256

    def body(y_ref, m_ref, o_ref):
        o_ref[0] = jnp.dot(y_ref[0], m_ref[...],
                           preferred_element_type=jnp.float32)

    return pl.pallas_call(
        body,
        grid=(B, BLK // TS),
        in_specs=[
            pl.BlockSpec((1, TS, D_MODEL), lambda b, i: (b, i, 0)),
            pl.BlockSpec((D_MODEL, D_MODEL), lambda b, i: (0, 0)),
        ],
        out_specs=pl.BlockSpec((1, TS, D_MODEL), lambda b, i: (b, i, 0)),
        out_shape=jax.ShapeDtypeStruct((B, BLK, D_MODEL), jnp.float32),
    )(y, merge)


def _sc_route(pT2, h2, B, S, K):
    E = N_EXP
    CH = 128
    NSLC = S // 16

    mesh = plsc.VectorSubcoreMesh(core_axis_name="c", subcore_axis_name="s")

    @functools.partial(
        pl.kernel,
        out_type=[
            jax.ShapeDtypeStruct((B * E * K, D_H), jnp.float32),
            jax.ShapeDtypeStruct((B * E, K), jnp.int32),
            jax.ShapeDtypeStruct((B * E, K), jnp.float32),
        ],
        mesh=mesh,
        scratch_types=[
            pltpu.VMEM((S,), jnp.float32),
            pltpu.VMEM((K + 16,), jnp.int32),
            pltpu.VMEM((K + 16,), jnp.int32),
            pltpu.VMEM((K + 16,), jnp.float32),
            pltpu.VMEM((S + 16,), jnp.int32),
            pltpu.VMEM((CH, D_H), jnp.float32),
            pltpu.SemaphoreType.DMA,
        ],
        compiler_params=pltpu.CompilerParams(needs_layout_passes=False),
    )
    def k(pT_hbm, h_hbm, xin_hbm, i_hbm, g_hbm,
          pbuf, gibuf, libuf, gbuf, tbuf, rows, sem):
        b = lax.axis_index("c")
        e = lax.axis_index("s")
        be = b * E + e
        bS = b * S
        pltpu.sync_copy(pT_hbm.at[be], pbuf)

        def count_ge(cand):
            def cbody(i, acc):
                def one(j, a):
                    v = pbuf[pl.ds((i * 4 + j) * 16, 16)]
                    vi = plsc.bitcast(v, jnp.int32)
                    return a + jnp.where(vi >= cand, 1, 0)
                return one(3, one(2, one(1, one(0, acc))))
            acc = lax.fori_loop(0, NSLC // 4, cbody,
                                jnp.zeros((16,), jnp.int32))
            return jnp.sum(acc)

        def bit_body(j, t):
            cand = t | lax.shift_left(jnp.int32(1), 29 - j)
            return jnp.where(count_ge(cand) >= K, cand, t)

        t = lax.fori_loop(0, 30, bit_body, jnp.int32(0))
        tf = lax.bitcast_convert_type(t, jnp.float32)

        def comp(i, carry):
            cnt, tcnt = carry
            base = i * 16
            v = pbuf[pl.ds(base, 16)]
            vi = plsc.bitcast(v, jnp.int32)
            idx = lax.iota(jnp.int32, 16) + base
            m_gt = vi > t
            plsc.store_compressed(libuf.at[pl.ds(cnt, 16)], idx, mask=m_gt)
            plsc.store_compressed(gibuf.at[pl.ds(cnt, 16)], idx + bS,
                                  mask=m_gt)
            plsc.store_compressed(gbuf.at[pl.ds(cnt, 16)], v, mask=m_gt)
            ngt = plsc.all_reduce_population_count(m_gt)[0]
            m_eq = vi == t
            plsc.store_compressed(tbuf.at[pl.ds(tcnt, 16)], idx, mask=m_eq)
            neq = plsc.all_reduce_population_count(m_eq)[0]
            return cnt + ngt, tcnt + neq

        cnt, _ = lax.fori_loop(0, NSLC, comp, (jnp.int32(0), jnp.int32(0)))

        r = K - cnt
        tfv = jnp.full((16,), tf, jnp.float32)

        def tie(i, _):
            base = i * 16
            lane = lax.iota(jnp.int32, 16) + base
            m = lane < r
            tv = tbuf[pl.ds(base, 16)]
            plsc.store_compressed(libuf.at[pl.ds(cnt + base, 16)], tv, mask=m)
            plsc.store_compressed(gibuf.at[pl.ds(cnt + base, 16)], tv + bS,
                                  mask=m)
            plsc.store_compressed(gbuf.at[pl.ds(cnt + base, 16)], tfv, mask=m)
            return jnp.int32(0)

        lax.fori_loop(0, (r + 15) // 16, tie, jnp.int32(0))

        pltpu.sync_copy(libuf.at[pl.ds(0, K)], i_hbm.at[be])
        pltpu.sync_copy(gbuf.at[pl.ds(0, K)], g_hbm.at[be])

        for c in range(K // CH):
            cp = pltpu.async_copy(h_hbm.at[gibuf.at[pl.ds(c * CH, CH)]],
                                  rows, sem)
            cp.wait()
            pltpu.sync_copy(rows, xin_hbm.at[pl.ds(be * K + c * CH, CH)])

    return k(pT2, h2)


def _sc_combine(contrib2, i2, B, S, K):
    E = N_EXP
    HALF = S // 2
    ZR = 40
    ACC = HALF + 128
    PER = ACC // E
    OUT = HALF // E
    CH = 128

    mesh = plsc.VectorSubcoreMesh(core_axis_name="c", subcore_axis_name="s")

    @functools.partial(
        pl.kernel,
        out_type=jax.ShapeDtypeStruct((B * S, D_H), jnp.float32),
        mesh=mesh,
        scratch_types=[
            pltpu.VMEM_SHARED((ACC, D_H), jnp.float32),
            pltpu.VMEM((K,), jnp.int32),
            pltpu.VMEM((CH,), jnp.int32),
            pltpu.VMEM((CH, D_H), jnp.float32),
            pltpu.VMEM((ZR, D_H), jnp.float32),
        ],
        compiler_params=pltpu.CompilerParams(needs_layout_passes=False),
    )
    def k(c_hbm, i_hbm, out_hbm, acc, ibuf, lbuf, cbuf, zbuf):
        b = lax.axis_index("c")
        e = lax.axis_index("s")
        be = b * E + e
        pltpu.sync_copy(i_hbm.at[be], ibuf)

        zv = jnp.zeros((16,), jnp.float32)

        def zfill(i, _):
            zr = i // (D_H // 16)
            zl = i % (D_H // 16)
            zbuf[zr, pl.ds(zl * 16, 16)] = zv
            return jnp.int32(0)

        lax.fori_loop(0, ZR * (D_H // 16), zfill, jnp.int32(0))

        for p in range(2):
            def zcopy(i, _):
                pltpu.sync_copy(zbuf, acc.at[pl.ds(e * PER + i * ZR, ZR)])
                return jnp.int32(0)

            lax.fori_loop(0, PER // ZR, zcopy, jnp.int32(0))
            plsc.subcore_barrier()

            def chunk(c, _):
                pltpu.sync_copy(c_hbm.at[pl.ds(be * K + c * CH, CH)], cbuf)
                for j in range(CH // 16):
                    idx = ibuf[pl.ds(c * CH + j * 16, 16)]
                    vi = idx - p * HALF
                    inr = (vi >= 0) & (vi < HALF)
                    lbuf[pl.ds(j * 16, 16)] = jnp.where(inr, vi, HALF)
                pltpu.sync_copy(cbuf, acc.at[lbuf], add=True)
                return jnp.int32(0)

            lax.fori_loop(0, K // CH, chunk, jnp.int32(0))
            plsc.subcore_barrier()
            pltpu.sync_copy(
                acc.at[pl.ds(e * OUT, OUT)],
                out_hbm.at[pl.ds(b * S + p * HALF + e * OUT, OUT)])
            plsc.subcore_barrier()

    return k(contrib2, i2)


def kernel(x, head, choice, w1, w2, merge):
    B, BLK, _ = x.shape
    S = BLK * N_HEAD
    K = S * CAP // N_EXP
    h, pT = _stage1(x, head, choice)
    h2 = h.reshape(B * S, D_H)
    pT2 = pT.reshape(B * N_EXP, S)
    xin2, i2, g2 = _sc_route(pT2, h2, B, S, K)
    xin = xin2.reshape(B, N_EXP, K, D_H)
    g = g2.reshape(B, N_EXP, K, 1)
    contrib = _tc_mlp(xin, w1, w2, g)
    out_moe = _sc_combine(contrib.reshape(B * N_EXP * K, D_H), i2, B, S, K)
    out = _tc_merge(out_moe.reshape(B, BLK, D_MODEL), merge)
    return out
